# Initial kernel scaffold; baseline (speedup 1.0000x reference)
import jax
import jax.numpy as jnp
from jax.experimental import pallas as pl


def kernel(x, edge_index, W1, b1, W2, b2, W3, b3):
    raise NotImplementedError("write your pallas kernel here")


# speedup vs baseline: 11.1993x; 11.1993x over previous
import functools

import jax
import jax.numpy as jnp
from jax import lax
from jax.experimental import pallas as pl
from jax.experimental.pallas import tpu as pltpu
from jax.experimental.pallas import tpu_sc as plsc

N = 50000
E = 800000
EP = 819200
NB_ROWS = EP // 128
TRASH = N
NPAD = 50048
RPTZ = NPAD // 16
RPT = 3136
RPT_LAST = N - 15 * RPT
NC, NS = 2, 16

@functools.lru_cache(None)
def _mesh():
    return plsc.VectorSubcoreMesh(core_axis_name="c", subcore_axis_name="s")


def _process_edges(tbl, src2, dst2, acc, sidx, didx, rows, gsem,
                   row0, nb, chunk):
    @pl.loop(0, nb // chunk)
    def _outer(g):
        r0 = row0 + g * chunk
        pltpu.sync_copy(src2.at[pl.ds(r0, chunk)], sidx)
        pltpu.sync_copy(dst2.at[pl.ds(r0, chunk)], didx)
        pltpu.async_copy(tbl.at[sidx.at[0]], rows.at[0], gsem)

        @pl.loop(0, chunk)
        def _inner(j):
            p = lax.rem(j, 2)

            @pl.when(j + 1 < chunk)
            def _():
                pltpu.async_copy(tbl.at[sidx.at[j + 1]], rows.at[1 - p], gsem)

            pltpu.make_async_copy(tbl.at[sidx.at[0]], rows.at[0], gsem).wait()
            pltpu.sync_copy(rows.at[p], acc.at[didx.at[j]], add=True)


def _copy_out(c, s, acc, out_a, out_b):
    sl_full = pl.ds(s * RPT, RPT)
    sl_last = pl.ds(15 * RPT, RPT_LAST)

    @pl.when(c == 0)
    def _():
        @pl.when(s < 15)
        def _():
            pltpu.sync_copy(acc.at[sl_full], out_a.at[sl_full])

        @pl.when(s == 15)
        def _():
            pltpu.sync_copy(acc.at[sl_last], out_a.at[sl_last])

    @pl.when(c == 1)
    def _():
        @pl.when(s < 15)
        def _():
            pltpu.sync_copy(acc.at[sl_full], out_b.at[sl_full])

        @pl.when(s == 15)
        def _():
            pltpu.sync_copy(acc.at[sl_last], out_b.at[sl_last])


def _make_seg_edge_split(w, nb_per_worker, chunk):
    @functools.partial(
        pl.kernel,
        out_type=[jax.ShapeDtypeStruct((N, w), jnp.float32)] * 2,
        mesh=_mesh(),
        scratch_types=[
            pltpu.VMEM_SHARED((NPAD, w), jnp.float32),
            pltpu.VMEM((chunk, 128), jnp.int32),
            pltpu.VMEM((chunk, 128), jnp.int32),
            pltpu.VMEM((2, 128, w), jnp.float32),
            pltpu.SemaphoreType.DMA,
        ],
        compiler_params=pltpu.CompilerParams(use_tc_tiling_on_sc=False),
    )
    def k(tbl, src2, dst2, zer, out_a, out_b, acc, sidx, didx, rows, gsem):
        c = lax.axis_index("c")
        s = lax.axis_index("s")
        pltpu.sync_copy(zer, acc.at[pl.ds(s * RPTZ, RPTZ)])
        plsc.subcore_barrier()
        wid = c * NS + s
        _process_edges(tbl, src2, dst2, acc, sidx, didx, rows, gsem,
                       wid * nb_per_worker, nb_per_worker, chunk)
        plsc.subcore_barrier()
        _copy_out(c, s, acc, out_a, out_b)

    return k


def _make_seg_feat_split(w, nb_per_tile, chunk):
    @functools.partial(
        pl.kernel,
        out_type=[jax.ShapeDtypeStruct((N, w), jnp.float32)] * 2,
        mesh=_mesh(),
        scratch_types=[
            pltpu.VMEM_SHARED((NPAD, w), jnp.float32),
            pltpu.VMEM((chunk, 128), jnp.int32),
            pltpu.VMEM((chunk, 128), jnp.int32),
            pltpu.VMEM((2, 128, w), jnp.float32),
            pltpu.SemaphoreType.DMA,
        ],
        compiler_params=pltpu.CompilerParams(use_tc_tiling_on_sc=False),
    )
    def k(tbl_a, tbl_b, src2, dst2, zer, out_a, out_b,
          acc, sidx, didx, rows, gsem):
        c = lax.axis_index("c")
        s = lax.axis_index("s")
        pltpu.sync_copy(zer, acc.at[pl.ds(s * RPTZ, RPTZ)])
        plsc.subcore_barrier()
        row0 = s * nb_per_tile

        @pl.when(c == 0)
        def _():
            _process_edges(tbl_a, src2, dst2, acc, sidx, didx, rows, gsem,
                           row0, nb_per_tile, chunk)

        @pl.when(c == 1)
        def _():
            _process_edges(tbl_b, src2, dst2, acc, sidx, didx, rows, gsem,
                           row0, nb_per_tile, chunk)

        plsc.subcore_barrier()
        _copy_out(c, s, acc, out_a, out_b)

    return k


_make_seg_edge_split = functools.lru_cache(None)(_make_seg_edge_split)
_make_seg_feat_split = functools.lru_cache(None)(_make_seg_feat_split)

_BLK = 400
_GRID = N // _BLK


def _tc_layer1(pa, pb, w1, b1r, w2a, w2b):
    def body(pa_ref, pb_ref, w1_ref, b1_ref, w2a_ref, w2b_ref,
             oa_ref, ob_ref):
        a0 = pa_ref[:, 0:1] + pb_ref[:, 0:1]
        a1 = pa_ref[:, 1:2] + pb_ref[:, 1:2]
        w1v = w1_ref[...]
        h = a0 * w1v[0:1, :] + a1 * w1v[1:2, :] + b1_ref[...]
        h = jnp.maximum(h, 0.0)
        oa_ref[...] = jnp.dot(h, w2a_ref[...],
                              preferred_element_type=jnp.float32)
        ob_ref[...] = jnp.dot(h, w2b_ref[...],
                              preferred_element_type=jnp.float32)

    return pl.pallas_call(
        body,
        grid=(_GRID,),
        in_specs=[
            pl.BlockSpec((_BLK, 16), lambda i: (i, 0)),
            pl.BlockSpec((_BLK, 16), lambda i: (i, 0)),
            pl.BlockSpec((2, 128), lambda i: (0, 0)),
            pl.BlockSpec((1, 128), lambda i: (0, 0)),
            pl.BlockSpec((128, 32), lambda i: (0, 0)),
            pl.BlockSpec((128, 32), lambda i: (0, 0)),
        ],
        out_specs=[pl.BlockSpec((_BLK, 32), lambda i: (i, 0))] * 2,
        out_shape=[jax.ShapeDtypeStruct((N, 32), jnp.float32)] * 2,
    )(pa, pb, w1, b1r, w2a, w2b)


def _tc_layer2(aa, ab, b2a, b2b, w3a, w3b):
    def body(aa_ref, ab_ref, b2a_ref, b2b_ref, w3a_ref, w3b_ref, o_ref):
        ha = jnp.maximum(aa_ref[...] + b2a_ref[...], 0.0)
        hb = jnp.maximum(ab_ref[...] + b2b_ref[...], 0.0)
        z = (jnp.sum(ha * w3a_ref[...], axis=1, keepdims=True)
             + jnp.sum(hb * w3b_ref[...], axis=1, keepdims=True))
        lane = lax.broadcasted_iota(jnp.int32, (1, 16), 1)
        o_ref[...] = jnp.where(lane == 0, z, 0.0)

    return pl.pallas_call(
        body,
        grid=(_GRID,),
        in_specs=[
            pl.BlockSpec((_BLK, 32), lambda i: (i, 0)),
            pl.BlockSpec((_BLK, 32), lambda i: (i, 0)),
            pl.BlockSpec((1, 32), lambda i: (0, 0)),
            pl.BlockSpec((1, 32), lambda i: (0, 0)),
            pl.BlockSpec((1, 32), lambda i: (0, 0)),
            pl.BlockSpec((1, 32), lambda i: (0, 0)),
        ],
        out_specs=pl.BlockSpec((_BLK, 16), lambda i: (i, 0)),
        out_shape=jax.ShapeDtypeStruct((N, 16), jnp.float32),
    )(aa, ab, b2a, b2b, w3a, w3b)


def _tc_final(pa, pb, b3r):
    def body(pa_ref, pb_ref, b3_ref, o_ref):
        o_ref[...] = pa_ref[:, 0:1] + pb_ref[:, 0:1] + b3_ref[...]

    return pl.pallas_call(
        body,
        grid=(_GRID,),
        in_specs=[
            pl.BlockSpec((_BLK, 16), lambda i: (i, 0)),
            pl.BlockSpec((_BLK, 16), lambda i: (i, 0)),
            pl.BlockSpec((1, 1), lambda i: (0, 0)),
        ],
        out_specs=pl.BlockSpec((_BLK, 1), lambda i: (i, 0)),
        out_shape=jax.ShapeDtypeStruct((N, 1), jnp.float32),
    )(pa, pb, b3r)


def kernel(x, edge_index, W1, b1, W2, b2, W3, b3):
    src = edge_index[0]
    dst = edge_index[1]
    pad = EP - E
    src2 = jnp.concatenate(
        [src, jnp.zeros((pad,), jnp.int32)]).reshape(NB_ROWS, 128)
    dst2 = jnp.concatenate(
        [dst, jnp.full((pad,), TRASH, jnp.int32)]).reshape(NB_ROWS, 128)
    zer16 = jnp.zeros((RPTZ, 16), jnp.float32)
    zer32 = jnp.zeros((RPTZ, 32), jnp.float32)

    xp = jnp.pad(x, ((0, 0), (0, 14)))
    p0a, p0b = _make_seg_edge_split(16, 200, 40)(xp, src2, dst2, zer16)
    z2a, z2b = _tc_layer1(p0a, p0b, W1, b1.reshape(1, 128),
                          W2[:, :32], W2[:, 32:])
    a2a, a2b = _make_seg_feat_split(32, 400, 40)(z2a, z2b, src2, dst2, zer32)
    z3 = _tc_layer2(a2a, a2b, b2[:32].reshape(1, 32), b2[32:].reshape(1, 32),
                    W3[:32, 0].reshape(1, 32), W3[32:, 0].reshape(1, 32))
    p3a, p3b = _make_seg_edge_split(16, 200, 40)(z3, src2, dst2, zer16)
    out = _tc_final(p3a, p3b, b3.reshape(1, 1))
    return jnp.squeeze(out, axis=-1)


# speedup vs baseline: 12.0147x; 1.0728x over previous
import functools

import jax
import jax.numpy as jnp
from jax import lax
from jax.experimental import pallas as pl
from jax.experimental.pallas import tpu as pltpu
from jax.experimental.pallas import tpu_sc as plsc

N = 50000
E = 800000
EP = 819200
NB_ROWS = EP // 128
TRASH = N
NPAD = 50048
RPTZ = NPAD // 16
RPT = 3136
RPT_LAST = N - 15 * RPT
NC, NS = 2, 16

@functools.lru_cache(None)
def _mesh():
    return plsc.VectorSubcoreMesh(core_axis_name="c", subcore_axis_name="s")


_K = 5
_K32 = 2
_CH = 40


def _process_edges(tbl, src2, dst2, acc, sidx, didx, rows, gsem, ssem,
                   row0, nb, chunk, k):
    ng = chunk // k

    @pl.loop(0, nb // chunk)
    def _outer(ci):
        r0 = row0 + ci * chunk
        pltpu.sync_copy(src2.at[pl.ds(r0, chunk)], sidx)
        pltpu.sync_copy(dst2.at[pl.ds(r0, chunk)], didx)
        for b in range(k):
            pltpu.async_copy(tbl.at[sidx.at[b]], rows.at[0, b],
                             gsem.at[0, b])

        @pl.loop(0, ng)
        def _group(g):
            p = lax.rem(g, 2)

            @pl.when(g + 1 < ng)
            def _():
                for b in range(k):
                    pltpu.async_copy(tbl.at[sidx.at[(g + 1) * k + b]],
                                     rows.at[1 - p, b], gsem.at[1 - p, b])

            for b in range(k):
                pltpu.make_async_copy(tbl.at[sidx.at[0]], rows.at[0, 0],
                                      gsem.at[p, b]).wait()
                pltpu.async_copy(rows.at[p, b], acc.at[didx.at[g * k + b]],
                                 ssem, add=True)
            for b in range(k):
                pltpu.make_async_copy(tbl.at[sidx.at[0]], rows.at[0, 0],
                                      ssem).wait()


def _copy_out(c, s, acc, out_a, out_b):
    sl_full = pl.ds(s * RPT, RPT)
    sl_last = pl.ds(15 * RPT, RPT_LAST)

    @pl.when(c == 0)
    def _():
        @pl.when(s < 15)
        def _():
            pltpu.sync_copy(acc.at[sl_full], out_a.at[sl_full])

        @pl.when(s == 15)
        def _():
            pltpu.sync_copy(acc.at[sl_last], out_a.at[sl_last])

    @pl.when(c == 1)
    def _():
        @pl.when(s < 15)
        def _():
            pltpu.sync_copy(acc.at[sl_full], out_b.at[sl_full])

        @pl.when(s == 15)
        def _():
            pltpu.sync_copy(acc.at[sl_last], out_b.at[sl_last])


def _make_seg_edge_split(w, nb_per_worker, chunk):
    @functools.partial(
        pl.kernel,
        out_type=[jax.ShapeDtypeStruct((N, w), jnp.float32)] * 2,
        mesh=_mesh(),
        scratch_types=[
            pltpu.VMEM_SHARED((NPAD, w), jnp.float32),
            pltpu.VMEM((_CH, 128), jnp.int32),
            pltpu.VMEM((_CH, 128), jnp.int32),
            pltpu.VMEM((2, _K, 128, w), jnp.float32),
            pltpu.SemaphoreType.DMA((2, _K)),
            pltpu.SemaphoreType.DMA,
        ],
        compiler_params=pltpu.CompilerParams(use_tc_tiling_on_sc=False),
    )
    def k(tbl, src2, dst2, zer, out_a, out_b,
          acc, sidx, didx, rows, gsem, ssem):
        c = lax.axis_index("c")
        s = lax.axis_index("s")
        pltpu.sync_copy(zer, acc.at[pl.ds(s * RPTZ, RPTZ)])
        plsc.subcore_barrier()
        wid = c * NS + s
        _process_edges(tbl, src2, dst2, acc, sidx, didx, rows, gsem, ssem,
                       wid * nb_per_worker, nb_per_worker, chunk, _K)
        plsc.subcore_barrier()
        _copy_out(c, s, acc, out_a, out_b)

    return k


def _make_seg_feat_split(w, nb_per_tile, chunk):
    @functools.partial(
        pl.kernel,
        out_type=[jax.ShapeDtypeStruct((N, w), jnp.float32)] * 2,
        mesh=_mesh(),
        scratch_types=[
            pltpu.VMEM_SHARED((NPAD, w), jnp.float32),
            pltpu.VMEM((_CH, 128), jnp.int32),
            pltpu.VMEM((_CH, 128), jnp.int32),
            pltpu.VMEM((2, _K32, 128, w), jnp.float32),
            pltpu.SemaphoreType.DMA((2, _K32)),
            pltpu.SemaphoreType.DMA,
        ],
        compiler_params=pltpu.CompilerParams(use_tc_tiling_on_sc=False),
    )
    def k(tbl_a, tbl_b, src2, dst2, zer, out_a, out_b,
          acc, sidx, didx, rows, gsem, ssem):
        c = lax.axis_index("c")
        s = lax.axis_index("s")
        pltpu.sync_copy(zer, acc.at[pl.ds(s * RPTZ, RPTZ)])
        plsc.subcore_barrier()
        row0 = s * nb_per_tile

        @pl.when(c == 0)
        def _():
            _process_edges(tbl_a, src2, dst2, acc, sidx, didx, rows,
                           gsem, ssem, row0, nb_per_tile, chunk, _K32)

        @pl.when(c == 1)
        def _():
            _process_edges(tbl_b, src2, dst2, acc, sidx, didx, rows,
                           gsem, ssem, row0, nb_per_tile, chunk, _K32)

        plsc.subcore_barrier()
        _copy_out(c, s, acc, out_a, out_b)

    return k


_make_seg_edge_split = functools.lru_cache(None)(_make_seg_edge_split)
_make_seg_feat_split = functools.lru_cache(None)(_make_seg_feat_split)

_BLK = 400
_GRID = N // _BLK


def _tc_layer1(pa, pb, w1, b1r, w2a, w2b):
    def body(pa_ref, pb_ref, w1_ref, b1_ref, w2a_ref, w2b_ref,
             oa_ref, ob_ref):
        a0 = pa_ref[:, 0:1] + pb_ref[:, 0:1]
        a1 = pa_ref[:, 1:2] + pb_ref[:, 1:2]
        w1v = w1_ref[...]
        h = a0 * w1v[0:1, :] + a1 * w1v[1:2, :] + b1_ref[...]
        h = jnp.maximum(h, 0.0)
        oa_ref[...] = jnp.dot(h, w2a_ref[...],
                              preferred_element_type=jnp.float32)
        ob_ref[...] = jnp.dot(h, w2b_ref[...],
                              preferred_element_type=jnp.float32)

    return pl.pallas_call(
        body,
        grid=(_GRID,),
        in_specs=[
            pl.BlockSpec((_BLK, 16), lambda i: (i, 0)),
            pl.BlockSpec((_BLK, 16), lambda i: (i, 0)),
            pl.BlockSpec((2, 128), lambda i: (0, 0)),
            pl.BlockSpec((1, 128), lambda i: (0, 0)),
            pl.BlockSpec((128, 32), lambda i: (0, 0)),
            pl.BlockSpec((128, 32), lambda i: (0, 0)),
        ],
        out_specs=[pl.BlockSpec((_BLK, 32), lambda i: (i, 0))] * 2,
        out_shape=[jax.ShapeDtypeStruct((N, 32), jnp.float32)] * 2,
    )(pa, pb, w1, b1r, w2a, w2b)


def _tc_layer2(aa, ab, b2a, b2b, w3a, w3b):
    def body(aa_ref, ab_ref, b2a_ref, b2b_ref, w3a_ref, w3b_ref, o_ref):
        ha = jnp.maximum(aa_ref[...] + b2a_ref[...], 0.0)
        hb = jnp.maximum(ab_ref[...] + b2b_ref[...], 0.0)
        z = (jnp.sum(ha * w3a_ref[...], axis=1, keepdims=True)
             + jnp.sum(hb * w3b_ref[...], axis=1, keepdims=True))
        lane = lax.broadcasted_iota(jnp.int32, (1, 16), 1)
        o_ref[...] = jnp.where(lane == 0, z, 0.0)

    return pl.pallas_call(
        body,
        grid=(_GRID,),
        in_specs=[
            pl.BlockSpec((_BLK, 32), lambda i: (i, 0)),
            pl.BlockSpec((_BLK, 32), lambda i: (i, 0)),
            pl.BlockSpec((1, 32), lambda i: (0, 0)),
            pl.BlockSpec((1, 32), lambda i: (0, 0)),
            pl.BlockSpec((1, 32), lambda i: (0, 0)),
            pl.BlockSpec((1, 32), lambda i: (0, 0)),
        ],
        out_specs=pl.BlockSpec((_BLK, 16), lambda i: (i, 0)),
        out_shape=jax.ShapeDtypeStruct((N, 16), jnp.float32),
    )(aa, ab, b2a, b2b, w3a, w3b)


def _tc_final(pa, pb, b3r):
    def body(pa_ref, pb_ref, b3_ref, o_ref):
        o_ref[...] = pa_ref[:, 0:1] + pb_ref[:, 0:1] + b3_ref[...]

    return pl.pallas_call(
        body,
        grid=(_GRID,),
        in_specs=[
            pl.BlockSpec((_BLK, 16), lambda i: (i, 0)),
            pl.BlockSpec((_BLK, 16), lambda i: (i, 0)),
            pl.BlockSpec((1, 1), lambda i: (0, 0)),
        ],
        out_specs=pl.BlockSpec((_BLK, 1), lambda i: (i, 0)),
        out_shape=jax.ShapeDtypeStruct((N, 1), jnp.float32),
    )(pa, pb, b3r)


def kernel(x, edge_index, W1, b1, W2, b2, W3, b3):
    src = edge_index[0]
    dst = edge_index[1]
    pad = EP - E
    src2 = jnp.concatenate(
        [src, jnp.zeros((pad,), jnp.int32)]).reshape(NB_ROWS, 128)
    dst2 = jnp.concatenate(
        [dst, jnp.full((pad,), TRASH, jnp.int32)]).reshape(NB_ROWS, 128)
    zer16 = jnp.zeros((RPTZ, 16), jnp.float32)
    zer32 = jnp.zeros((RPTZ, 32), jnp.float32)

    xp = jnp.pad(x, ((0, 0), (0, 14)))
    p0a, p0b = _make_seg_edge_split(16, 200, 40)(xp, src2, dst2, zer16)
    z2a, z2b = _tc_layer1(p0a, p0b, W1, b1.reshape(1, 128),
                          W2[:, :32], W2[:, 32:])
    a2a, a2b = _make_seg_feat_split(32, 400, 40)(z2a, z2b, src2, dst2, zer32)
    z3 = _tc_layer2(a2a, a2b, b2[:32].reshape(1, 32), b2[32:].reshape(1, 32),
                    W3[:32, 0].reshape(1, 32), W3[32:, 0].reshape(1, 32))
    p3a, p3b = _make_seg_edge_split(16, 200, 40)(z3, src2, dst2, zer16)
    out = _tc_final(p3a, p3b, b3.reshape(1, 1))
    return jnp.squeeze(out, axis=-1)


# speedup vs baseline: 12.0386x; 1.0020x over previous
import functools

import jax
import jax.numpy as jnp
from jax import lax
from jax.experimental import pallas as pl
from jax.experimental.pallas import tpu as pltpu
from jax.experimental.pallas import tpu_sc as plsc

N = 50000
E = 800000
EP = 819200
NB_ROWS = EP // 128
NPAD = 51200
RPTZ = NPAD // 16
RPT = 3136
RPT_LAST = N - 15 * RPT
NC, NS = 2, 16

@functools.lru_cache(None)
def _mesh():
    return plsc.VectorSubcoreMesh(core_axis_name="c", subcore_axis_name="s")


_K = 5
_K32 = 2
_CH = 40


def _process_edges(tbl, src2, dst2, acc, sidx, didx, rows, gsem, ssem,
                   row0, nb, chunk, k):
    ng = chunk // k

    @pl.loop(0, nb // chunk)
    def _outer(ci):
        r0 = row0 + ci * chunk
        pltpu.sync_copy(src2.at[pl.ds(r0, chunk)], sidx)
        pltpu.sync_copy(dst2.at[pl.ds(r0, chunk)], didx)
        for b in range(k):
            pltpu.async_copy(tbl.at[sidx.at[b]], rows.at[0, b],
                             gsem.at[0, b])

        @pl.loop(0, ng)
        def _group(g):
            p = lax.rem(g, 2)

            @pl.when(g + 1 < ng)
            def _():
                for b in range(k):
                    pltpu.async_copy(tbl.at[sidx.at[(g + 1) * k + b]],
                                     rows.at[1 - p, b], gsem.at[1 - p, b])

            for b in range(k):
                pltpu.make_async_copy(tbl.at[sidx.at[0]], rows.at[0, 0],
                                      gsem.at[p, b]).wait()
                pltpu.async_copy(rows.at[p, b], acc.at[didx.at[g * k + b]],
                                 ssem, add=True)
            for b in range(k):
                pltpu.make_async_copy(tbl.at[sidx.at[0]], rows.at[0, 0],
                                      ssem).wait()


def _copy_out(c, s, acc, out_a, out_b):
    sl_full = pl.ds(s * RPT, RPT)
    sl_last = pl.ds(15 * RPT, RPT_LAST)

    @pl.when(c == 0)
    def _():
        @pl.when(s < 15)
        def _():
            pltpu.sync_copy(acc.at[sl_full], out_a.at[sl_full])

        @pl.when(s == 15)
        def _():
            pltpu.sync_copy(acc.at[sl_last], out_a.at[sl_last])

    @pl.when(c == 1)
    def _():
        @pl.when(s < 15)
        def _():
            pltpu.sync_copy(acc.at[sl_full], out_b.at[sl_full])

        @pl.when(s == 15)
        def _():
            pltpu.sync_copy(acc.at[sl_last], out_b.at[sl_last])


def _make_seg_edge_split(w, nb_per_worker, chunk):
    @functools.partial(
        pl.kernel,
        out_type=[jax.ShapeDtypeStruct((N, w), jnp.float32)] * 2,
        mesh=_mesh(),
        scratch_types=[
            pltpu.VMEM_SHARED((NPAD, w), jnp.float32),
            pltpu.VMEM((_CH, 128), jnp.int32),
            pltpu.VMEM((_CH, 128), jnp.int32),
            pltpu.VMEM((2, _K, 128, w), jnp.float32),
            pltpu.SemaphoreType.DMA((2, _K)),
            pltpu.SemaphoreType.DMA,
        ],
        compiler_params=pltpu.CompilerParams(use_tc_tiling_on_sc=False),
    )
    def k(tbl, src2, dst2, zer, out_a, out_b,
          acc, sidx, didx, rows, gsem, ssem):
        c = lax.axis_index("c")
        s = lax.axis_index("s")
        pltpu.sync_copy(zer, acc.at[pl.ds(s * RPTZ, RPTZ)])
        plsc.subcore_barrier()
        wid = c * NS + s
        _process_edges(tbl, src2, dst2, acc, sidx, didx, rows, gsem, ssem,
                       wid * nb_per_worker, nb_per_worker, chunk, _K)
        plsc.subcore_barrier()
        _copy_out(c, s, acc, out_a, out_b)

    return k


def _make_seg_feat_split(w, nb_per_tile, chunk):
    @functools.partial(
        pl.kernel,
        out_type=[jax.ShapeDtypeStruct((N, w), jnp.float32)] * 2,
        mesh=_mesh(),
        scratch_types=[
            pltpu.VMEM_SHARED((NPAD, w), jnp.float32),
            pltpu.VMEM((_CH, 128), jnp.int32),
            pltpu.VMEM((_CH, 128), jnp.int32),
            pltpu.VMEM((2, _K32, 128, w), jnp.float32),
            pltpu.SemaphoreType.DMA((2, _K32)),
            pltpu.SemaphoreType.DMA,
        ],
        compiler_params=pltpu.CompilerParams(use_tc_tiling_on_sc=False),
    )
    def k(tbl_a, tbl_b, src2, dst2, zer, out_a, out_b,
          acc, sidx, didx, rows, gsem, ssem):
        c = lax.axis_index("c")
        s = lax.axis_index("s")
        pltpu.sync_copy(zer, acc.at[pl.ds(s * RPTZ, RPTZ)])
        plsc.subcore_barrier()
        row0 = s * nb_per_tile

        @pl.when(c == 0)
        def _():
            _process_edges(tbl_a, src2, dst2, acc, sidx, didx, rows,
                           gsem, ssem, row0, nb_per_tile, chunk, _K32)

        @pl.when(c == 1)
        def _():
            _process_edges(tbl_b, src2, dst2, acc, sidx, didx, rows,
                           gsem, ssem, row0, nb_per_tile, chunk, _K32)

        plsc.subcore_barrier()
        _copy_out(c, s, acc, out_a, out_b)

    return k


_make_seg_edge_split = functools.lru_cache(None)(_make_seg_edge_split)
_make_seg_feat_split = functools.lru_cache(None)(_make_seg_feat_split)

_BLK = 400
_GRID = N // _BLK


def _tc_layer1(pa, pb, w1, b1r, w2a, w2b):
    def body(pa_ref, pb_ref, w1_ref, b1_ref, w2a_ref, w2b_ref,
             oa_ref, ob_ref):
        a0 = pa_ref[:, 0:1] + pb_ref[:, 0:1]
        a1 = pa_ref[:, 1:2] + pb_ref[:, 1:2]
        w1v = w1_ref[...]
        h = a0 * w1v[0:1, :] + a1 * w1v[1:2, :] + b1_ref[...]
        h = jnp.maximum(h, 0.0)
        oa_ref[...] = jnp.dot(h, w2a_ref[...],
                              preferred_element_type=jnp.float32)
        ob_ref[...] = jnp.dot(h, w2b_ref[...],
                              preferred_element_type=jnp.float32)

    return pl.pallas_call(
        body,
        grid=(_GRID,),
        in_specs=[
            pl.BlockSpec((_BLK, 16), lambda i: (i, 0)),
            pl.BlockSpec((_BLK, 16), lambda i: (i, 0)),
            pl.BlockSpec((2, 128), lambda i: (0, 0)),
            pl.BlockSpec((1, 128), lambda i: (0, 0)),
            pl.BlockSpec((128, 32), lambda i: (0, 0)),
            pl.BlockSpec((128, 32), lambda i: (0, 0)),
        ],
        out_specs=[pl.BlockSpec((_BLK, 32), lambda i: (i, 0))] * 2,
        out_shape=[jax.ShapeDtypeStruct((N, 32), jnp.float32)] * 2,
    )(pa, pb, w1, b1r, w2a, w2b)


def _tc_layer2(aa, ab, b2a, b2b, w3a, w3b):
    def body(aa_ref, ab_ref, b2a_ref, b2b_ref, w3a_ref, w3b_ref, o_ref):
        ha = jnp.maximum(aa_ref[...] + b2a_ref[...], 0.0)
        hb = jnp.maximum(ab_ref[...] + b2b_ref[...], 0.0)
        z = (jnp.sum(ha * w3a_ref[...], axis=1, keepdims=True)
             + jnp.sum(hb * w3b_ref[...], axis=1, keepdims=True))
        lane = lax.broadcasted_iota(jnp.int32, (1, 16), 1)
        o_ref[...] = jnp.where(lane == 0, z, 0.0)

    return pl.pallas_call(
        body,
        grid=(_GRID,),
        in_specs=[
            pl.BlockSpec((_BLK, 32), lambda i: (i, 0)),
            pl.BlockSpec((_BLK, 32), lambda i: (i, 0)),
            pl.BlockSpec((1, 32), lambda i: (0, 0)),
            pl.BlockSpec((1, 32), lambda i: (0, 0)),
            pl.BlockSpec((1, 32), lambda i: (0, 0)),
            pl.BlockSpec((1, 32), lambda i: (0, 0)),
        ],
        out_specs=pl.BlockSpec((_BLK, 16), lambda i: (i, 0)),
        out_shape=jax.ShapeDtypeStruct((N, 16), jnp.float32),
    )(aa, ab, b2a, b2b, w3a, w3b)


def _tc_final(pa, pb, b3r):
    def body(pa_ref, pb_ref, b3_ref, o_ref):
        o_ref[...] = pa_ref[:, 0:1] + pb_ref[:, 0:1] + b3_ref[...]

    return pl.pallas_call(
        body,
        grid=(_GRID,),
        in_specs=[
            pl.BlockSpec((_BLK, 16), lambda i: (i, 0)),
            pl.BlockSpec((_BLK, 16), lambda i: (i, 0)),
            pl.BlockSpec((1, 1), lambda i: (0, 0)),
        ],
        out_specs=pl.BlockSpec((_BLK, 1), lambda i: (i, 0)),
        out_shape=jax.ShapeDtypeStruct((N, 1), jnp.float32),
    )(pa, pb, b3r)


def kernel(x, edge_index, W1, b1, W2, b2, W3, b3):
    src = edge_index[0]
    dst = edge_index[1]
    pad = EP - E
    src2 = jnp.concatenate(
        [src, jnp.zeros((pad,), jnp.int32)]).reshape(NB_ROWS, 128)
    trash = N + jnp.arange(pad, dtype=jnp.int32) % (NPAD - N)
    dst2 = jnp.concatenate([dst, trash]).reshape(NB_ROWS, 128)
    zer16 = jnp.zeros((RPTZ, 16), jnp.float32)
    zer32 = jnp.zeros((RPTZ, 32), jnp.float32)

    xp = jnp.pad(x, ((0, 0), (0, 14)))
    p0a, p0b = _make_seg_edge_split(16, 200, 40)(xp, src2, dst2, zer16)
    z2a, z2b = _tc_layer1(p0a, p0b, W1, b1.reshape(1, 128),
                          W2[:, :32], W2[:, 32:])
    a2a, a2b = _make_seg_feat_split(32, 400, 40)(z2a, z2b, src2, dst2, zer32)
    z3 = _tc_layer2(a2a, a2b, b2[:32].reshape(1, 32), b2[32:].reshape(1, 32),
                    W3[:32, 0].reshape(1, 32), W3[32:, 0].reshape(1, 32))
    p3a, p3b = _make_seg_edge_split(16, 200, 40)(z3, src2, dst2, zer16)
    out = _tc_final(p3a, p3b, b3.reshape(1, 1))
    return jnp.squeeze(out, axis=-1)


# speedup vs baseline: 17.9830x; 1.4938x over previous
import functools

import jax
import jax.numpy as jnp
from jax import lax
from jax.experimental import pallas as pl
from jax.experimental.pallas import tpu as pltpu
from jax.experimental.pallas import tpu_sc as plsc

N = 50000
E = 800000
EP = 819200
NB_ROWS = EP // 128
NPAD = 51200
RPTZ = NPAD // 16
RPT = 3136
RPT_LAST = N - 15 * RPT
NC, NS = 2, 16

@functools.lru_cache(None)
def _mesh():
    return plsc.VectorSubcoreMesh(core_axis_name="c", subcore_axis_name="s")


_K = 5
_K32 = 2
_CH = 40


def _process_edges(tbl, src2, dst2, acc, sidx, didx, rows, gsem, ssem,
                   row0, nb, chunk, k):
    ng = chunk // k

    @pl.loop(0, nb // chunk)
    def _outer(ci):
        r0 = row0 + ci * chunk
        pltpu.sync_copy(src2.at[pl.ds(r0, chunk)], sidx)
        pltpu.sync_copy(dst2.at[pl.ds(r0, chunk)], didx)
        for b in range(k):
            pltpu.async_copy(tbl.at[sidx.at[b]], rows.at[0, b],
                             gsem.at[0, b])

        @pl.loop(0, ng)
        def _group(g):
            p = lax.rem(g, 2)

            @pl.when(g + 1 < ng)
            def _():
                for b in range(k):
                    pltpu.async_copy(tbl.at[sidx.at[(g + 1) * k + b]],
                                     rows.at[1 - p, b], gsem.at[1 - p, b])

            for b in range(k):
                pltpu.make_async_copy(tbl.at[sidx.at[0]], rows.at[0, 0],
                                      gsem.at[p, b]).wait()
                pltpu.async_copy(rows.at[p, b], acc.at[didx.at[g * k + b]],
                                 ssem, add=True)
            for b in range(k):
                pltpu.make_async_copy(tbl.at[sidx.at[0]], rows.at[0, 0],
                                      ssem).wait()


def _copy_out(c, s, acc, out_a, out_b):
    sl_full = pl.ds(s * RPT, RPT)
    sl_last = pl.ds(15 * RPT, RPT_LAST)

    @pl.when(c == 0)
    def _():
        @pl.when(s < 15)
        def _():
            pltpu.sync_copy(acc.at[sl_full], out_a.at[sl_full])

        @pl.when(s == 15)
        def _():
            pltpu.sync_copy(acc.at[sl_last], out_a.at[sl_last])

    @pl.when(c == 1)
    def _():
        @pl.when(s < 15)
        def _():
            pltpu.sync_copy(acc.at[sl_full], out_b.at[sl_full])

        @pl.when(s == 15)
        def _():
            pltpu.sync_copy(acc.at[sl_last], out_b.at[sl_last])


def _make_seg_edge_split(w, nb_per_worker, chunk):
    @functools.partial(
        pl.kernel,
        out_type=[jax.ShapeDtypeStruct((N, w), jnp.float32)] * 2,
        mesh=_mesh(),
        scratch_types=[
            pltpu.VMEM_SHARED((NPAD, w), jnp.float32),
            pltpu.VMEM((_CH, 128), jnp.int32),
            pltpu.VMEM((_CH, 128), jnp.int32),
            pltpu.VMEM((2, _K, 128, w), jnp.float32),
            pltpu.SemaphoreType.DMA((2, _K)),
            pltpu.SemaphoreType.DMA,
        ],
        compiler_params=pltpu.CompilerParams(use_tc_tiling_on_sc=False),
    )
    def k(tbl, src2, dst2, zer, out_a, out_b,
          acc, sidx, didx, rows, gsem, ssem):
        c = lax.axis_index("c")
        s = lax.axis_index("s")
        pltpu.sync_copy(zer, acc.at[pl.ds(s * RPTZ, RPTZ)])
        plsc.subcore_barrier()
        wid = c * NS + s
        _process_edges(tbl, src2, dst2, acc, sidx, didx, rows, gsem, ssem,
                       wid * nb_per_worker, nb_per_worker, chunk, _K)
        plsc.subcore_barrier()
        _copy_out(c, s, acc, out_a, out_b)

    return k


def _make_seg_feat_split(w, nb_per_tile, chunk):
    @functools.partial(
        pl.kernel,
        out_type=[jax.ShapeDtypeStruct((N, w), jnp.float32)] * 2,
        mesh=_mesh(),
        scratch_types=[
            pltpu.VMEM_SHARED((NPAD, w), jnp.float32),
            pltpu.VMEM((_CH, 128), jnp.int32),
            pltpu.VMEM((_CH, 128), jnp.int32),
            pltpu.VMEM((2, _K32, 128, w), jnp.float32),
            pltpu.SemaphoreType.DMA((2, _K32)),
            pltpu.SemaphoreType.DMA,
        ],
        compiler_params=pltpu.CompilerParams(use_tc_tiling_on_sc=False),
    )
    def k(tbl_a, tbl_b, src2, dst2, zer, out_a, out_b,
          acc, sidx, didx, rows, gsem, ssem):
        c = lax.axis_index("c")
        s = lax.axis_index("s")
        pltpu.sync_copy(zer, acc.at[pl.ds(s * RPTZ, RPTZ)])
        plsc.subcore_barrier()
        row0 = s * nb_per_tile

        @pl.when(c == 0)
        def _():
            _process_edges(tbl_a, src2, dst2, acc, sidx, didx, rows,
                           gsem, ssem, row0, nb_per_tile, chunk, _K32)

        @pl.when(c == 1)
        def _():
            _process_edges(tbl_b, src2, dst2, acc, sidx, didx, rows,
                           gsem, ssem, row0, nb_per_tile, chunk, _K32)

        plsc.subcore_barrier()
        _copy_out(c, s, acc, out_a, out_b)

    return k


_make_seg_edge_split = functools.lru_cache(None)(_make_seg_edge_split)
_make_seg_feat_split = functools.lru_cache(None)(_make_seg_feat_split)

_BLK = 400
_GRID = N // _BLK


def _tc_layer1(pa, pb, w1, b1r, w2a, w2b):
    def body(pa_ref, pb_ref, w1_ref, b1_ref, w2a_ref, w2b_ref,
             oa_ref, ob_ref):
        a0 = pa_ref[:, 0:1] + pb_ref[:, 0:1]
        a1 = pa_ref[:, 1:2] + pb_ref[:, 1:2]
        w1v = w1_ref[...]
        h = a0 * w1v[0:1, :] + a1 * w1v[1:2, :] + b1_ref[...]
        h = jnp.maximum(h, 0.0)
        oa_ref[...] = jnp.dot(h, w2a_ref[...],
                              preferred_element_type=jnp.float32)
        ob_ref[...] = jnp.dot(h, w2b_ref[...],
                              preferred_element_type=jnp.float32)

    return pl.pallas_call(
        body,
        grid=(_GRID,),
        in_specs=[
            pl.BlockSpec((_BLK, 16), lambda i: (i, 0)),
            pl.BlockSpec((_BLK, 16), lambda i: (i, 0)),
            pl.BlockSpec((2, 128), lambda i: (0, 0)),
            pl.BlockSpec((1, 128), lambda i: (0, 0)),
            pl.BlockSpec((128, 32), lambda i: (0, 0)),
            pl.BlockSpec((128, 32), lambda i: (0, 0)),
        ],
        out_specs=[pl.BlockSpec((_BLK, 32), lambda i: (i, 0))] * 2,
        out_shape=[jax.ShapeDtypeStruct((N, 32), jnp.float32)] * 2,
    )(pa, pb, w1, b1r, w2a, w2b)


def _tc_layer2(aa, ab, b2a, b2b, w3a, w3b):
    def body(aa_ref, ab_ref, b2a_ref, b2b_ref, w3a_ref, w3b_ref, o_ref):
        ha = jnp.maximum(aa_ref[...] + b2a_ref[...], 0.0)
        hb = jnp.maximum(ab_ref[...] + b2b_ref[...], 0.0)
        z = (jnp.sum(ha * w3a_ref[...], axis=1, keepdims=True)
             + jnp.sum(hb * w3b_ref[...], axis=1, keepdims=True))
        lane = lax.broadcasted_iota(jnp.int32, (1, 16), 1)
        o_ref[...] = jnp.where(lane == 0, z, 0.0)

    return pl.pallas_call(
        body,
        grid=(_GRID,),
        in_specs=[
            pl.BlockSpec((_BLK, 32), lambda i: (i, 0)),
            pl.BlockSpec((_BLK, 32), lambda i: (i, 0)),
            pl.BlockSpec((1, 32), lambda i: (0, 0)),
            pl.BlockSpec((1, 32), lambda i: (0, 0)),
            pl.BlockSpec((1, 32), lambda i: (0, 0)),
            pl.BlockSpec((1, 32), lambda i: (0, 0)),
        ],
        out_specs=pl.BlockSpec((_BLK, 16), lambda i: (i, 0)),
        out_shape=jax.ShapeDtypeStruct((N, 16), jnp.float32),
    )(aa, ab, b2a, b2b, w3a, w3b)


def _tc_final(pa, pb, b3r):
    def body(pa_ref, pb_ref, b3_ref, o_ref):
        o_ref[...] = pa_ref[:, 0:1] + pb_ref[:, 0:1] + b3_ref[...]

    return pl.pallas_call(
        body,
        grid=(_GRID,),
        in_specs=[
            pl.BlockSpec((_BLK, 16), lambda i: (i, 0)),
            pl.BlockSpec((_BLK, 16), lambda i: (i, 0)),
            pl.BlockSpec((1, 1), lambda i: (0, 0)),
        ],
        out_specs=pl.BlockSpec((_BLK, 1), lambda i: (i, 0)),
        out_shape=jax.ShapeDtypeStruct((N, 1), jnp.float32),
    )(pa, pb, b3r)


def kernel(x, edge_index, W1, b1, W2, b2, W3, b3):
    src = edge_index[0]
    dst = edge_index[1]
    pad = EP - E
    psrc = jnp.arange(pad, dtype=jnp.int32) % N
    src2 = jnp.concatenate([src, psrc]).reshape(NB_ROWS, 128)
    trash = N + jnp.arange(pad, dtype=jnp.int32) % (NPAD - N)
    dst2 = jnp.concatenate([dst, trash]).reshape(NB_ROWS, 128)
    zer16 = jnp.zeros((RPTZ, 16), jnp.float32)
    zer32 = jnp.zeros((RPTZ, 32), jnp.float32)

    xp = jnp.pad(x, ((0, 0), (0, 14)))
    p0a, p0b = _make_seg_edge_split(16, 200, 40)(xp, src2, dst2, zer16)
    z2a, z2b = _tc_layer1(p0a, p0b, W1, b1.reshape(1, 128),
                          W2[:, :32], W2[:, 32:])
    a2a, a2b = _make_seg_feat_split(32, 400, 40)(z2a, z2b, src2, dst2, zer32)
    z3 = _tc_layer2(a2a, a2b, b2[:32].reshape(1, 32), b2[32:].reshape(1, 32),
                    W3[:32, 0].reshape(1, 32), W3[32:, 0].reshape(1, 32))
    p3a, p3b = _make_seg_edge_split(16, 200, 40)(z3, src2, dst2, zer16)
    out = _tc_final(p3a, p3b, b3.reshape(1, 1))
    return jnp.squeeze(out, axis=-1)


# speedup vs baseline: 22.8049x; 1.2681x over previous
import functools

import jax
import jax.numpy as jnp
from jax import lax
from jax.experimental import pallas as pl
from jax.experimental.pallas import tpu as pltpu
from jax.experimental.pallas import tpu_sc as plsc

N = 50000
E = 800000
EP = 819200
NB_ROWS = EP // 128
NPAD = 51200
RPTZ = NPAD // 16
RPT = 3136
RPT_LAST = N - 15 * RPT
NC, NS = 2, 16

@functools.lru_cache(None)
def _mesh():
    return plsc.VectorSubcoreMesh(core_axis_name="c", subcore_axis_name="s")


_K = 5
_K32 = 2
_CH = 40


def _process_edges(tbl, src2, dst2, acc, sidx, didx, rows, gsem, ssem,
                   row0, nb, chunk, k):
    ng = chunk // k

    @pl.loop(0, nb // chunk)
    def _outer(ci):
        r0 = row0 + ci * chunk
        pltpu.sync_copy(src2.at[pl.ds(r0, chunk)], sidx)
        pltpu.sync_copy(dst2.at[pl.ds(r0, chunk)], didx)
        for b in range(k):
            pltpu.async_copy(tbl.at[sidx.at[b]], rows.at[0, b],
                             gsem.at[0, b])

        @pl.loop(0, ng)
        def _group(g):
            p = lax.rem(g, 2)

            @pl.when(g + 1 < ng)
            def _():
                for b in range(k):
                    pltpu.async_copy(tbl.at[sidx.at[(g + 1) * k + b]],
                                     rows.at[1 - p, b], gsem.at[1 - p, b])

            for b in range(k):
                pltpu.make_async_copy(tbl.at[sidx.at[0]], rows.at[0, 0],
                                      gsem.at[p, b]).wait()
                pltpu.async_copy(rows.at[p, b], acc.at[didx.at[g * k + b]],
                                 ssem, add=True)
            for b in range(k):
                pltpu.make_async_copy(tbl.at[sidx.at[0]], rows.at[0, 0],
                                      ssem).wait()


def _copy_out(c, s, acc, out_a, out_b):
    sl_full = pl.ds(s * RPT, RPT)
    sl_last = pl.ds(15 * RPT, RPT_LAST)

    @pl.when(c == 0)
    def _():
        @pl.when(s < 15)
        def _():
            pltpu.sync_copy(acc.at[sl_full], out_a.at[sl_full])

        @pl.when(s == 15)
        def _():
            pltpu.sync_copy(acc.at[sl_last], out_a.at[sl_last])

    @pl.when(c == 1)
    def _():
        @pl.when(s < 15)
        def _():
            pltpu.sync_copy(acc.at[sl_full], out_b.at[sl_full])

        @pl.when(s == 15)
        def _():
            pltpu.sync_copy(acc.at[sl_last], out_b.at[sl_last])


def _make_seg_edge_split(w, nb_per_worker, chunk):
    @functools.partial(
        pl.kernel,
        out_type=[jax.ShapeDtypeStruct((N, w), jnp.float32)] * 2,
        mesh=_mesh(),
        scratch_types=[
            pltpu.VMEM_SHARED((NPAD, w), jnp.float32),
            pltpu.VMEM((_CH, 128), jnp.int32),
            pltpu.VMEM((_CH, 128), jnp.int32),
            pltpu.VMEM((2, _K, 128, w), jnp.float32),
            pltpu.SemaphoreType.DMA((2, _K)),
            pltpu.SemaphoreType.DMA,
        ],
        compiler_params=pltpu.CompilerParams(use_tc_tiling_on_sc=False),
    )
    def k(tbl, src2, dst2, zer, out_a, out_b,
          acc, sidx, didx, rows, gsem, ssem):
        c = lax.axis_index("c")
        s = lax.axis_index("s")
        pltpu.sync_copy(zer, acc.at[pl.ds(s * RPTZ, RPTZ)])
        plsc.subcore_barrier()
        wid = c * NS + s
        _process_edges(tbl, src2, dst2, acc, sidx, didx, rows, gsem, ssem,
                       wid * nb_per_worker, nb_per_worker, chunk, _K)
        plsc.subcore_barrier()
        _copy_out(c, s, acc, out_a, out_b)

    return k


def _make_seg_feat_split(w, nb_per_tile, chunk):
    @functools.partial(
        pl.kernel,
        out_type=[jax.ShapeDtypeStruct((N, w), jnp.float32)] * 2,
        mesh=_mesh(),
        scratch_types=[
            pltpu.VMEM_SHARED((NPAD, w), jnp.float32),
            pltpu.VMEM((_CH, 128), jnp.int32),
            pltpu.VMEM((_CH, 128), jnp.int32),
            pltpu.VMEM((2, _K32, 128, w), jnp.float32),
            pltpu.SemaphoreType.DMA((2, _K32)),
            pltpu.SemaphoreType.DMA,
        ],
        compiler_params=pltpu.CompilerParams(use_tc_tiling_on_sc=False),
    )
    def k(tbl_a, tbl_b, src2, dst2, zer, out_a, out_b,
          acc, sidx, didx, rows, gsem, ssem):
        c = lax.axis_index("c")
        s = lax.axis_index("s")
        pltpu.sync_copy(zer, acc.at[pl.ds(s * RPTZ, RPTZ)])
        plsc.subcore_barrier()
        row0 = s * nb_per_tile

        @pl.when(c == 0)
        def _():
            _process_edges(tbl_a, src2, dst2, acc, sidx, didx, rows,
                           gsem, ssem, row0, nb_per_tile, chunk, _K32)

        @pl.when(c == 1)
        def _():
            _process_edges(tbl_b, src2, dst2, acc, sidx, didx, rows,
                           gsem, ssem, row0, nb_per_tile, chunk, _K32)

        plsc.subcore_barrier()
        _copy_out(c, s, acc, out_a, out_b)

    return k


_make_seg_edge_split = functools.lru_cache(None)(_make_seg_edge_split)
_make_seg_feat_split = functools.lru_cache(None)(_make_seg_feat_split)

_BLK = 5000
_GRID = N // _BLK


def _tc_layer1(pa, pb, w1, b1r, w2a, w2b):
    def body(pa_ref, pb_ref, w1_ref, b1_ref, w2a_ref, w2b_ref,
             oa_ref, ob_ref):
        a0 = pa_ref[:, 0:1] + pb_ref[:, 0:1]
        a1 = pa_ref[:, 1:2] + pb_ref[:, 1:2]
        w1v = w1_ref[...]
        h = a0 * w1v[0:1, :] + a1 * w1v[1:2, :] + b1_ref[...]
        h = jnp.maximum(h, 0.0)
        oa_ref[...] = jnp.dot(h, w2a_ref[...],
                              preferred_element_type=jnp.float32)
        ob_ref[...] = jnp.dot(h, w2b_ref[...],
                              preferred_element_type=jnp.float32)

    return pl.pallas_call(
        body,
        grid=(_GRID,),
        in_specs=[
            pl.BlockSpec((_BLK, 16), lambda i: (i, 0)),
            pl.BlockSpec((_BLK, 16), lambda i: (i, 0)),
            pl.BlockSpec((2, 128), lambda i: (0, 0)),
            pl.BlockSpec((1, 128), lambda i: (0, 0)),
            pl.BlockSpec((128, 32), lambda i: (0, 0)),
            pl.BlockSpec((128, 32), lambda i: (0, 0)),
        ],
        out_specs=[pl.BlockSpec((_BLK, 32), lambda i: (i, 0))] * 2,
        out_shape=[jax.ShapeDtypeStruct((N, 32), jnp.float32)] * 2,
    )(pa, pb, w1, b1r, w2a, w2b)


def _tc_layer2(aa, ab, b2a, b2b, w3a, w3b):
    def body(aa_ref, ab_ref, b2a_ref, b2b_ref, w3a_ref, w3b_ref, o_ref):
        ha = jnp.maximum(aa_ref[...] + b2a_ref[...], 0.0)
        hb = jnp.maximum(ab_ref[...] + b2b_ref[...], 0.0)
        z = (jnp.sum(ha * w3a_ref[...], axis=1, keepdims=True)
             + jnp.sum(hb * w3b_ref[...], axis=1, keepdims=True))
        o_ref[...] = jnp.broadcast_to(z, (_BLK, 16))

    return pl.pallas_call(
        body,
        grid=(_GRID,),
        in_specs=[
            pl.BlockSpec((_BLK, 32), lambda i: (i, 0)),
            pl.BlockSpec((_BLK, 32), lambda i: (i, 0)),
            pl.BlockSpec((1, 32), lambda i: (0, 0)),
            pl.BlockSpec((1, 32), lambda i: (0, 0)),
            pl.BlockSpec((1, 32), lambda i: (0, 0)),
            pl.BlockSpec((1, 32), lambda i: (0, 0)),
        ],
        out_specs=pl.BlockSpec((_BLK, 16), lambda i: (i, 0)),
        out_shape=jax.ShapeDtypeStruct((N, 16), jnp.float32),
    )(aa, ab, b2a, b2b, w3a, w3b)


def _tc_final(pa, pb, b3r):
    def body(pa_ref, pb_ref, b3_ref, o_ref):
        o_ref[...] = pa_ref[:, 0:1] + pb_ref[:, 0:1] + b3_ref[...]

    return pl.pallas_call(
        body,
        grid=(_GRID,),
        in_specs=[
            pl.BlockSpec((_BLK, 16), lambda i: (i, 0)),
            pl.BlockSpec((_BLK, 16), lambda i: (i, 0)),
            pl.BlockSpec((1, 1), lambda i: (0, 0)),
        ],
        out_specs=pl.BlockSpec((_BLK, 1), lambda i: (i, 0)),
        out_shape=jax.ShapeDtypeStruct((N, 1), jnp.float32),
    )(pa, pb, b3r)


def kernel(x, edge_index, W1, b1, W2, b2, W3, b3):
    src = edge_index[0]
    dst = edge_index[1]
    pad = EP - E
    psrc = jnp.arange(pad, dtype=jnp.int32) % N
    src2 = jnp.concatenate([src, psrc]).reshape(NB_ROWS, 128)
    trash = N + jnp.arange(pad, dtype=jnp.int32) % (NPAD - N)
    dst2 = jnp.concatenate([dst, trash]).reshape(NB_ROWS, 128)
    zer16 = jnp.zeros((RPTZ, 16), jnp.float32)
    zer32 = jnp.zeros((RPTZ, 32), jnp.float32)

    xp = jnp.pad(x, ((0, 0), (0, 14)))
    p0a, p0b = _make_seg_edge_split(16, 200, 40)(xp, src2, dst2, zer16)
    z2a, z2b = _tc_layer1(p0a, p0b, W1, b1.reshape(1, 128),
                          W2[:, :32], W2[:, 32:])
    a2a, a2b = _make_seg_feat_split(32, 400, 40)(z2a, z2b, src2, dst2, zer32)
    z3 = _tc_layer2(a2a, a2b, b2[:32].reshape(1, 32), b2[32:].reshape(1, 32),
                    W3[:32, 0].reshape(1, 32), W3[32:, 0].reshape(1, 32))
    p3a, p3b = _make_seg_edge_split(16, 200, 40)(z3, src2, dst2, zer16)
    out = _tc_final(p3a, p3b, b3.reshape(1, 1))
    return jnp.squeeze(out, axis=-1)


# speedup vs baseline: 24.7925x; 1.0872x over previous
import functools

import jax
import jax.numpy as jnp
from jax import lax
from jax.experimental import pallas as pl
from jax.experimental.pallas import tpu as pltpu
from jax.experimental.pallas import tpu_sc as plsc

N = 50000
E = 800000
EP = 819200
NB_ROWS = EP // 128
NPAD = 50432
RPTZ = NPAD // 16
RPT = 3136
RPT_LAST = N - 15 * RPT
NC, NS = 2, 16

@functools.lru_cache(None)
def _mesh():
    return plsc.VectorSubcoreMesh(core_axis_name="c", subcore_axis_name="s")


_NS = 8
_LA = 4
_NS32 = 6
_LA32 = 3
_CH = 40
_CH32 = 16


def _process_edges(tbl, src2, dst2, acc, sidx, didx, rows, gsem, ssem,
                   row0, nb, chunk, ns, la):
    @pl.loop(0, nb // chunk)
    def _outer(ci):
        r0 = row0 + ci * chunk
        pltpu.sync_copy(src2.at[pl.ds(r0, chunk)], sidx)
        pltpu.sync_copy(dst2.at[pl.ds(r0, chunk)], didx)
        for q in range(la):
            pltpu.async_copy(tbl.at[sidx.at[q]], rows.at[q], gsem.at[q])

        @pl.loop(0, chunk)
        def _it(j):
            sl = lax.rem(j, ns)
            nxt = j + la

            @pl.when(nxt < chunk)
            def _():
                lsl = lax.rem(nxt, ns)

                @pl.when(nxt >= ns)
                def _():
                    pltpu.make_async_copy(tbl.at[sidx.at[0]], rows.at[0],
                                          ssem.at[lsl]).wait()

                pltpu.async_copy(tbl.at[sidx.at[nxt]], rows.at[lsl],
                                 gsem.at[lsl])

            pltpu.make_async_copy(tbl.at[sidx.at[0]], rows.at[0],
                                  gsem.at[sl]).wait()
            pltpu.async_copy(rows.at[sl], acc.at[didx.at[j]],
                             ssem.at[sl], add=True)

        for q in range(ns):
            pltpu.make_async_copy(tbl.at[sidx.at[0]], rows.at[0],
                                  ssem.at[q]).wait()


def _copy_out(c, s, acc, out_a, out_b):
    sl_full = pl.ds(s * RPT, RPT)
    sl_last = pl.ds(15 * RPT, RPT_LAST)

    @pl.when(c == 0)
    def _():
        @pl.when(s < 15)
        def _():
            pltpu.sync_copy(acc.at[sl_full], out_a.at[sl_full])

        @pl.when(s == 15)
        def _():
            pltpu.sync_copy(acc.at[sl_last], out_a.at[sl_last])

    @pl.when(c == 1)
    def _():
        @pl.when(s < 15)
        def _():
            pltpu.sync_copy(acc.at[sl_full], out_b.at[sl_full])

        @pl.when(s == 15)
        def _():
            pltpu.sync_copy(acc.at[sl_last], out_b.at[sl_last])


def _make_seg_edge_split(w, nb_per_worker, chunk):
    @functools.partial(
        pl.kernel,
        out_type=[jax.ShapeDtypeStruct((N, w), jnp.float32)] * 2,
        mesh=_mesh(),
        scratch_types=[
            pltpu.VMEM_SHARED((NPAD, w), jnp.float32),
            pltpu.VMEM((_CH, 128), jnp.int32),
            pltpu.VMEM((_CH, 128), jnp.int32),
            pltpu.VMEM((_NS, 128, w), jnp.float32),
            pltpu.SemaphoreType.DMA((_NS,)),
            pltpu.SemaphoreType.DMA((_NS,)),
        ],
        compiler_params=pltpu.CompilerParams(use_tc_tiling_on_sc=False),
    )
    def k(tbl, src2, dst2, zer, out_a, out_b,
          acc, sidx, didx, rows, gsem, ssem):
        c = lax.axis_index("c")
        s = lax.axis_index("s")
        pltpu.sync_copy(zer, acc.at[pl.ds(s * RPTZ, RPTZ)])
        plsc.subcore_barrier()
        wid = c * NS + s
        _process_edges(tbl, src2, dst2, acc, sidx, didx, rows, gsem, ssem,
                       wid * nb_per_worker, nb_per_worker, chunk, _NS, _LA)
        plsc.subcore_barrier()
        _copy_out(c, s, acc, out_a, out_b)

    return k


def _make_seg_feat_split(w, nb_per_tile, chunk):
    @functools.partial(
        pl.kernel,
        out_type=[jax.ShapeDtypeStruct((N, w), jnp.float32)] * 2,
        mesh=_mesh(),
        scratch_types=[
            pltpu.VMEM_SHARED((NPAD, w), jnp.float32),
            pltpu.VMEM((_CH32, 128), jnp.int32),
            pltpu.VMEM((_CH32, 128), jnp.int32),
            pltpu.VMEM((_NS32, 128, w), jnp.float32),
            pltpu.SemaphoreType.DMA((_NS32,)),
            pltpu.SemaphoreType.DMA((_NS32,)),
        ],
        compiler_params=pltpu.CompilerParams(use_tc_tiling_on_sc=False),
    )
    def k(tbl_a, tbl_b, src2, dst2, zer, out_a, out_b,
          acc, sidx, didx, rows, gsem, ssem):
        c = lax.axis_index("c")
        s = lax.axis_index("s")
        pltpu.sync_copy(zer, acc.at[pl.ds(s * RPTZ, RPTZ)])
        plsc.subcore_barrier()
        row0 = s * nb_per_tile

        @pl.when(c == 0)
        def _():
            _process_edges(tbl_a, src2, dst2, acc, sidx, didx, rows, gsem,
                           ssem, row0, nb_per_tile, _CH32, _NS32, _LA32)

        @pl.when(c == 1)
        def _():
            _process_edges(tbl_b, src2, dst2, acc, sidx, didx, rows, gsem,
                           ssem, row0, nb_per_tile, _CH32, _NS32, _LA32)

        plsc.subcore_barrier()
        _copy_out(c, s, acc, out_a, out_b)

    return k


_make_seg_edge_split = functools.lru_cache(None)(_make_seg_edge_split)
_make_seg_feat_split = functools.lru_cache(None)(_make_seg_feat_split)

_BLK = 5000
_GRID = N // _BLK


def _tc_layer1(pa, pb, w1, b1r, w2a, w2b):
    def body(pa_ref, pb_ref, w1_ref, b1_ref, w2a_ref, w2b_ref,
             oa_ref, ob_ref):
        a0 = pa_ref[:, 0:1] + pb_ref[:, 0:1]
        a1 = pa_ref[:, 1:2] + pb_ref[:, 1:2]
        w1v = w1_ref[...]
        h = a0 * w1v[0:1, :] + a1 * w1v[1:2, :] + b1_ref[...]
        h = jnp.maximum(h, 0.0)
        oa_ref[...] = jnp.dot(h, w2a_ref[...],
                              preferred_element_type=jnp.float32)
        ob_ref[...] = jnp.dot(h, w2b_ref[...],
                              preferred_element_type=jnp.float32)

    return pl.pallas_call(
        body,
        grid=(_GRID,),
        in_specs=[
            pl.BlockSpec((_BLK, 16), lambda i: (i, 0)),
            pl.BlockSpec((_BLK, 16), lambda i: (i, 0)),
            pl.BlockSpec((2, 128), lambda i: (0, 0)),
            pl.BlockSpec((1, 128), lambda i: (0, 0)),
            pl.BlockSpec((128, 32), lambda i: (0, 0)),
            pl.BlockSpec((128, 32), lambda i: (0, 0)),
        ],
        out_specs=[pl.BlockSpec((_BLK, 32), lambda i: (i, 0))] * 2,
        out_shape=[jax.ShapeDtypeStruct((N, 32), jnp.float32)] * 2,
    )(pa, pb, w1, b1r, w2a, w2b)


def _tc_layer2(aa, ab, b2a, b2b, w3a, w3b):
    def body(aa_ref, ab_ref, b2a_ref, b2b_ref, w3a_ref, w3b_ref, o_ref):
        ha = jnp.maximum(aa_ref[...] + b2a_ref[...], 0.0)
        hb = jnp.maximum(ab_ref[...] + b2b_ref[...], 0.0)
        z = (jnp.sum(ha * w3a_ref[...], axis=1, keepdims=True)
             + jnp.sum(hb * w3b_ref[...], axis=1, keepdims=True))
        o_ref[...] = jnp.broadcast_to(z, (_BLK, 16))

    return pl.pallas_call(
        body,
        grid=(_GRID,),
        in_specs=[
            pl.BlockSpec((_BLK, 32), lambda i: (i, 0)),
            pl.BlockSpec((_BLK, 32), lambda i: (i, 0)),
            pl.BlockSpec((1, 32), lambda i: (0, 0)),
            pl.BlockSpec((1, 32), lambda i: (0, 0)),
            pl.BlockSpec((1, 32), lambda i: (0, 0)),
            pl.BlockSpec((1, 32), lambda i: (0, 0)),
        ],
        out_specs=pl.BlockSpec((_BLK, 16), lambda i: (i, 0)),
        out_shape=jax.ShapeDtypeStruct((N, 16), jnp.float32),
    )(aa, ab, b2a, b2b, w3a, w3b)


_RC = 1568
_RC_LAST = N - 31 * _RC


@functools.lru_cache(None)
def _make_combine():
    @functools.partial(
        pl.kernel,
        out_type=jax.ShapeDtypeStruct((N,), jnp.float32),
        mesh=_mesh(),
        scratch_types=[
            pltpu.VMEM((_RC, 16), jnp.float32),
            pltpu.VMEM((_RC, 16), jnp.float32),
            pltpu.VMEM((_RC,), jnp.float32),
            pltpu.VMEM((16,), jnp.float32),
        ],
        compiler_params=pltpu.CompilerParams(use_tc_tiling_on_sc=False,
                                             needs_layout_passes=False),
    )
    def k(pa, pb, b3v, out, va, vb, vo, vbb):
        c = lax.axis_index("c")
        s = lax.axis_index("s")
        wid = c * NS + s
        base = wid * _RC
        pltpu.sync_copy(b3v, vbb)

        def run(r):
            pltpu.sync_copy(pa.at[pl.ds(base, r)], va.at[pl.ds(0, r)])
            pltpu.sync_copy(pb.at[pl.ds(base, r)], vb.at[pl.ds(0, r)])
            bb = vbb[...]
            zc = jnp.zeros((16,), jnp.int32)

            @pl.loop(0, r // 16)
            def _(i):
                ridx = lax.iota(jnp.int32, 16) + i * 16
                g = (plsc.load_gather(va, [ridx, zc])
                     + plsc.load_gather(vb, [ridx, zc]) + bb)
                vo[pl.ds(i * 16, 16)] = g

            pltpu.sync_copy(vo.at[pl.ds(0, r)], out.at[pl.ds(base, r)])

        @pl.when(wid < 31)
        def _():
            run(_RC)

        @pl.when(wid == 31)
        def _():
            run(_RC_LAST)

    return k


def kernel(x, edge_index, W1, b1, W2, b2, W3, b3):
    src = edge_index[0]
    dst = edge_index[1]
    pad = EP - E
    psrc = jnp.arange(pad, dtype=jnp.int32) % N
    src2 = jnp.concatenate([src, psrc]).reshape(NB_ROWS, 128)
    trash = N + jnp.arange(pad, dtype=jnp.int32) % (NPAD - N)
    dst2 = jnp.concatenate([dst, trash]).reshape(NB_ROWS, 128)
    zer16 = jnp.zeros((RPTZ, 16), jnp.float32)
    zer32 = jnp.zeros((RPTZ, 32), jnp.float32)

    xp = jnp.pad(x, ((0, 0), (0, 14)))
    p0a, p0b = _make_seg_edge_split(16, 200, 40)(xp, src2, dst2, zer16)
    z2a, z2b = _tc_layer1(p0a, p0b, W1, b1.reshape(1, 128),
                          W2[:, :32], W2[:, 32:])
    a2a, a2b = _make_seg_feat_split(32, 400, 40)(z2a, z2b, src2, dst2, zer32)
    z3 = _tc_layer2(a2a, a2b, b2[:32].reshape(1, 32), b2[32:].reshape(1, 32),
                    W3[:32, 0].reshape(1, 32), W3[32:, 0].reshape(1, 32))
    p3a, p3b = _make_seg_edge_split(16, 200, 40)(z3, src2, dst2, zer16)
    return _make_combine()(p3a, p3b, jnp.broadcast_to(b3, (16,)))
